# baseline (speedup 1.0000x reference)
import functools

import jax
import jax.numpy as jnp
from jax import lax
from jax.experimental import pallas as pl
from jax.experimental.pallas import tpu as pltpu
from jax.experimental.pallas import tpu_sc as plsc

N_NODES = 10000
N_EDGES = 160000
D = 256

NC = 2
NS = 16
L = 16

N_HALF = N_NODES // NC
ROWS_ACC = 5008
TRASH = N_HALF
ECHUNK = 1280
GCHUNK = 128
NG = ECHUNK // GCHUNK
NGRP = ECHUNK // L


NCH = N_EDGES // ECHUNK


def _sc_body(src_hbm, dst_hbm, xr_hbm, out_hbm,
             acc, srcbuf, dstbuf, rows, sem_st, sem_g0, sem_g1):
    sem_g = [sem_g0, sem_g1]
    c = lax.axis_index("c")
    s = lax.axis_index("s")
    half_base = c * N_HALF
    hbv = jnp.broadcast_to(half_base, (L,))
    trash_v = jnp.full((L,), TRASH, jnp.int32)
    xr_s = xr_hbm.at[s]

    def _z(i, carry):
        acc[i, pl.ds(0, L)] = jnp.zeros((L,), jnp.float32)
        return carry
    lax.fori_loop(0, ROWS_ACC, _z, 0)

    def _stage(j, b):
        e0 = j * ECHUNK
        pltpu.async_copy(src_hbm.at[pl.ds(e0, ECHUNK)], srcbuf.at[b], sem_st)
        pltpu.async_copy(dst_hbm.at[pl.ds(e0, ECHUNK)], dstbuf.at[b], sem_st)

    def _wait_stage(b):
        pltpu.make_async_copy(src_hbm.at[pl.ds(0, ECHUNK)],
                              srcbuf.at[b], sem_st).wait()
        pltpu.make_async_copy(dst_hbm.at[pl.ds(0, ECHUNK)],
                              dstbuf.at[b], sem_st).wait()

    def _fire_gathers(b):
        for k in range(NG):
            pltpu.async_copy(
                xr_s.at[srcbuf.at[b, pl.ds(k * GCHUNK, GCHUNK)]],
                rows.at[b, pl.ds(k * GCHUNK, GCHUNK)],
                sem_g[b],
            )

    def _wait_gathers(b):
        for k in range(NG):
            pltpu.make_async_copy(
                xr_s.at[srcbuf.at[b, pl.ds(k * GCHUNK, GCHUNK)]],
                rows.at[b, pl.ds(k * GCHUNK, GCHUNK)],
                sem_g[b],
            ).wait()

    def _accumulate(b):
        @plsc.parallel_loop(0, NGRP, unroll=4)
        def _grp(g):
            d = dstbuf[b, pl.ds(g * L, L)]
            m = (d >= hbv) & (d < hbv + N_HALF)
            dl = jnp.where(m, d - hbv, trash_v)
            for l in range(L):
                r = rows[b, g * L + l, pl.ds(0, L)]
                plsc.addupdate(acc.at[dl[l], pl.ds(0, L)], r)

    _stage(0, 0)
    _wait_stage(0)
    _fire_gathers(0)
    _stage(1, 1)

    def _steady(t, carry):
        j = 2 * t
        _wait_stage(1)
        _fire_gathers(1)
        _wait_gathers(0)
        _accumulate(0)

        @pl.when(j + 2 < NCH)
        def _():
            _stage(j + 2, 0)
            _wait_stage(0)
            _fire_gathers(0)

        _wait_gathers(1)
        _accumulate(1)

        @pl.when(j + 3 < NCH)
        def _():
            _stage(j + 3, 1)
        return carry
    lax.fori_loop(0, NCH // 2, _steady, 0)

    _wait_gathers(0)
    _accumulate(0)

    pltpu.sync_copy(
        acc.at[pl.ds(0, N_HALF)],
        out_hbm.at[pl.ds(c * N_HALF, N_HALF), s],
    )


_sc_agg = functools.partial(
    pl.kernel,
    mesh=plsc.VectorSubcoreMesh(core_axis_name="c", subcore_axis_name="s"),
    out_type=jax.ShapeDtypeStruct((N_NODES, NS, L), jnp.float32),
    compiler_params=pltpu.CompilerParams(use_tc_tiling_on_sc=False),
    scratch_types=[
        pltpu.VMEM((ROWS_ACC, L), jnp.float32),
        pltpu.VMEM((2, ECHUNK), jnp.int32),
        pltpu.VMEM((2, ECHUNK), jnp.int32),
        pltpu.VMEM((2, ECHUNK, L), jnp.float32),
        pltpu.SemaphoreType.DMA,
        pltpu.SemaphoreType.DMA,
        pltpu.SemaphoreType.DMA,
    ],
)(_sc_body)


ROWS_TC = 1000


def _tc_body(x_ref, agg_ref, ws_ref, wn_ref, b_ref, o_ref):
    z = jnp.dot(x_ref[...], ws_ref[...], preferred_element_type=jnp.float32)
    z = z + jnp.dot(agg_ref[...], wn_ref[...], preferred_element_type=jnp.float32)
    z = z + b_ref[...]
    o_ref[...] = jnp.where(z >= 0, z, jnp.float32(0.01) * z)


def _tc_dense(x, agg, w_self, w_nbr, bias):
    return pl.pallas_call(
        _tc_body,
        grid=(N_NODES // ROWS_TC,),
        in_specs=[
            pl.BlockSpec((ROWS_TC, D), lambda b: (b, 0)),
            pl.BlockSpec((ROWS_TC, D), lambda b: (b, 0)),
            pl.BlockSpec((D, D), lambda b: (0, 0)),
            pl.BlockSpec((D, D), lambda b: (0, 0)),
            pl.BlockSpec((1, D), lambda b: (0, 0)),
        ],
        out_specs=pl.BlockSpec((ROWS_TC, D), lambda b: (b, 0)),
        out_shape=jax.ShapeDtypeStruct((N_NODES, D), jnp.float32),
    )(x, agg, w_self, w_nbr, bias)


def kernel(x, edge_index, W_self, W_nbr, b):
    src = edge_index[0].astype(jnp.int32)
    dst = edge_index[1].astype(jnp.int32)
    xr = x.reshape(N_NODES, NS, L).transpose(1, 0, 2)
    agg = _sc_agg(src, dst, xr).reshape(N_NODES, D)
    return _tc_dense(x, agg, W_self, W_nbr, b.reshape(1, D))


# speedup vs baseline: 1.0912x; 1.0912x over previous
import functools

import jax
import jax.numpy as jnp
from jax import lax
from jax.experimental import pallas as pl
from jax.experimental.pallas import tpu as pltpu
from jax.experimental.pallas import tpu_sc as plsc

N_NODES = 10000
N_EDGES = 160000
D = 256

NC = 2
NS = 16
L = 16

N_HALF = N_NODES // NC
ROWS_ACC = 5008
TRASH = N_HALF
ECHUNK = 1280
GCHUNK = 128
NG = ECHUNK // GCHUNK
NGRP = ECHUNK // L


NCH = N_EDGES // ECHUNK


def _sc_body(src_hbm, dst_hbm, xr_hbm, out_hbm,
             acc, srcbuf, dstbuf, rows, sem_st, sem_g0, sem_g1):
    sem_g = [sem_g0, sem_g1]
    c = lax.axis_index("c")
    s = lax.axis_index("s")
    half_base = c * N_HALF
    hbv = jnp.broadcast_to(half_base, (L,))
    trash_v = jnp.full((L,), TRASH, jnp.int32)
    xr_s = xr_hbm.at[s]

    def _z(i, carry):
        acc[i, pl.ds(0, L)] = jnp.zeros((L,), jnp.float32)
        return carry
    lax.fori_loop(0, ROWS_ACC, _z, 0)

    def _stage(j, b):
        e0 = j * ECHUNK
        pltpu.async_copy(src_hbm.at[pl.ds(e0, ECHUNK)], srcbuf.at[b], sem_st)
        pltpu.async_copy(dst_hbm.at[pl.ds(e0, ECHUNK)], dstbuf.at[b], sem_st)

    def _wait_stage(b):
        pltpu.make_async_copy(src_hbm.at[pl.ds(0, ECHUNK)],
                              srcbuf.at[b], sem_st).wait()
        pltpu.make_async_copy(dst_hbm.at[pl.ds(0, ECHUNK)],
                              dstbuf.at[b], sem_st).wait()

    def _fire_gathers(b):
        for k in range(NG):
            pltpu.async_copy(
                xr_s.at[srcbuf.at[b, pl.ds(k * GCHUNK, GCHUNK)]],
                rows.at[b, pl.ds(k * GCHUNK, GCHUNK)],
                sem_g[b],
            )

    def _wait_gathers(b):
        for k in range(NG):
            pltpu.make_async_copy(
                xr_s.at[srcbuf.at[b, pl.ds(k * GCHUNK, GCHUNK)]],
                rows.at[b, pl.ds(k * GCHUNK, GCHUNK)],
                sem_g[b],
            ).wait()

    def _accumulate(b):
        @plsc.parallel_loop(0, NGRP, unroll=2)
        def _grp(g):
            d = dstbuf[b, pl.ds(g * L, L)]
            m = (d >= hbv) & (d < hbv + N_HALF)
            dl = jnp.where(m, d - hbv, trash_v)
            for l in range(L):
                r = rows[b, g * L + l, pl.ds(0, L)]
                plsc.addupdate(acc.at[dl[l], pl.ds(0, L)], r)

    _stage(0, 0)
    _wait_stage(0)
    _fire_gathers(0)
    _stage(1, 1)

    def _steady(t, carry):
        j = 2 * t
        _wait_stage(1)
        _fire_gathers(1)
        _wait_gathers(0)

        @pl.when(j + 2 < NCH)
        def _():
            _stage(j + 2, 0)

        _accumulate(0)

        @pl.when(j + 2 < NCH)
        def _():
            _wait_stage(0)
            _fire_gathers(0)

        _wait_gathers(1)

        @pl.when(j + 3 < NCH)
        def _():
            _stage(j + 3, 1)

        _accumulate(1)
        return carry
    lax.fori_loop(0, NCH // 2, _steady, 0)

    _wait_gathers(0)
    _accumulate(0)

    pltpu.sync_copy(
        acc.at[pl.ds(0, N_HALF)],
        out_hbm.at[pl.ds(c * N_HALF, N_HALF), s],
    )


_sc_agg = functools.partial(
    pl.kernel,
    mesh=plsc.VectorSubcoreMesh(core_axis_name="c", subcore_axis_name="s"),
    out_type=jax.ShapeDtypeStruct((N_NODES, NS, L), jnp.float32),
    compiler_params=pltpu.CompilerParams(use_tc_tiling_on_sc=False),
    scratch_types=[
        pltpu.VMEM((ROWS_ACC, L), jnp.float32),
        pltpu.VMEM((2, ECHUNK), jnp.int32),
        pltpu.VMEM((2, ECHUNK), jnp.int32),
        pltpu.VMEM((2, ECHUNK, L), jnp.float32),
        pltpu.SemaphoreType.DMA,
        pltpu.SemaphoreType.DMA,
        pltpu.SemaphoreType.DMA,
    ],
)(_sc_body)


ROWS_TC = 1000


def _tc_body(x_ref, agg_ref, ws_ref, wn_ref, b_ref, o_ref):
    z = jnp.dot(x_ref[...], ws_ref[...], preferred_element_type=jnp.float32)
    z = z + jnp.dot(agg_ref[...], wn_ref[...], preferred_element_type=jnp.float32)
    z = z + b_ref[...]
    o_ref[...] = jnp.where(z >= 0, z, jnp.float32(0.01) * z)


def _tc_dense(x, agg, w_self, w_nbr, bias):
    return pl.pallas_call(
        _tc_body,
        grid=(N_NODES // ROWS_TC,),
        in_specs=[
            pl.BlockSpec((ROWS_TC, D), lambda b: (b, 0)),
            pl.BlockSpec((ROWS_TC, D), lambda b: (b, 0)),
            pl.BlockSpec((D, D), lambda b: (0, 0)),
            pl.BlockSpec((D, D), lambda b: (0, 0)),
            pl.BlockSpec((1, D), lambda b: (0, 0)),
        ],
        out_specs=pl.BlockSpec((ROWS_TC, D), lambda b: (b, 0)),
        out_shape=jax.ShapeDtypeStruct((N_NODES, D), jnp.float32),
    )(x, agg, w_self, w_nbr, bias)


def kernel(x, edge_index, W_self, W_nbr, b):
    src = edge_index[0].astype(jnp.int32)
    dst = edge_index[1].astype(jnp.int32)
    xr = x.reshape(N_NODES, NS, L).transpose(1, 0, 2)
    agg = _sc_agg(src, dst, xr).reshape(N_NODES, D)
    return _tc_dense(x, agg, W_self, W_nbr, b.reshape(1, D))
